# baseline (device time: 14483 ns/iter reference)
import jax
import jax.numpy as jnp
from jax import lax
from jax.experimental import pallas as pl
from jax.experimental.pallas import tpu as pltpu

N_DEV = 8
E_PER = 2
E_TOT = N_DEV * E_PER
D = 128
H = 256


def kernel(x, router_W, route_idx, expert_W):
    T, _ = x.shape

    def body(x_ref, rw_ref, idx_ref, ew_ref, out_ref,
             gather_ref, send_sems, recv_sems):
        my = lax.axis_index("i")

        gather_ref[pl.ds(my * E_PER, E_PER)] = ew_ref[...].astype(jnp.bfloat16)

        barrier = pltpu.get_barrier_semaphore()
        for d in range(1, N_DEV):
            tgt = lax.rem(my + d, N_DEV)
            pl.semaphore_signal(
                barrier, inc=1,
                device_id=(tgt,), device_id_type=pl.DeviceIdType.MESH,
            )
        pl.semaphore_wait(barrier, N_DEV - 1)

        sends = []
        for d in range(1, N_DEV):
            tgt = lax.rem(my + d, N_DEV)
            rdma = pltpu.make_async_remote_copy(
                src_ref=gather_ref.at[pl.ds(my * E_PER, E_PER)],
                dst_ref=gather_ref.at[pl.ds(my * E_PER, E_PER)],
                send_sem=send_sems.at[d - 1],
                recv_sem=recv_sems.at[my],
                device_id=(tgt,),
                device_id_type=pl.DeviceIdType.MESH,
            )
            rdma.start()
            sends.append(rdma)

        x_f32 = x_ref[...]
        scores = jnp.dot(x_f32, rw_ref[...], preferred_element_type=jnp.float32)
        m = jnp.max(scores, axis=-1, keepdims=True)
        p = jnp.exp(scores - m)
        p = p / jnp.sum(p, axis=-1, keepdims=True)
        e_iota = lax.broadcasted_iota(jnp.int32, (T, E_TOT), 1)
        mask = (e_iota == idx_ref[:, 0:1]) | (e_iota == idx_ref[:, 1:2])
        w = jnp.where(mask, p, 0.0)
        w = w / jnp.sum(w, axis=-1, keepdims=True)

        for d in range(1, N_DEV):
            src_dev = lax.rem(my - d + N_DEV, N_DEV)
            recv = pltpu.make_async_remote_copy(
                src_ref=gather_ref.at[pl.ds(src_dev * E_PER, E_PER)],
                dst_ref=gather_ref.at[pl.ds(src_dev * E_PER, E_PER)],
                send_sem=send_sems.at[d - 1],
                recv_sem=recv_sems.at[src_dev],
                device_id=(src_dev,),
                device_id_type=pl.DeviceIdType.MESH,
            )
            recv.wait_recv()

        acc = jnp.zeros((T, H), jnp.float32)
        for e in range(E_TOT):
            xe = (x_f32 * w[:, e:e + 1]).astype(jnp.bfloat16)
            acc = acc + jnp.dot(xe, gather_ref[e],
                                preferred_element_type=jnp.float32)
        out_ref[...] = acc

        for rdma in sends:
            rdma.wait_send()

    return pl.pallas_call(
        body,
        out_shape=jax.ShapeDtypeStruct((T, H), jnp.float32),
        in_specs=[
            pl.BlockSpec(memory_space=pltpu.VMEM),
            pl.BlockSpec(memory_space=pltpu.VMEM),
            pl.BlockSpec(memory_space=pltpu.VMEM),
            pl.BlockSpec(memory_space=pltpu.VMEM),
        ],
        out_specs=pl.BlockSpec(memory_space=pltpu.VMEM),
        scratch_shapes=[
            pltpu.VMEM((E_TOT, D, H), jnp.bfloat16),
            pltpu.SemaphoreType.DMA((N_DEV - 1,)),
            pltpu.SemaphoreType.DMA((N_DEV,)),
        ],
        compiler_params=pltpu.CompilerParams(collective_id=0),
    )(x, router_W, route_idx, expert_W)
